# baseline (speedup 1.0000x reference)
import functools

import jax
import jax.numpy as jnp
from jax import lax
from jax.experimental import pallas as pl
from jax.experimental.pallas import tpu as pltpu
from jax.experimental.pallas import tpu_sc as plsc

_BATCH = 16384
_SUBLEN = 50
_EMBED = 64
_VOCAB = 1000000
_PAD = 128
_NC = 2
_NS = 16
_NW = _NC * _NS
_ROWS_PER_W = _BATCH // _NW
_CHUNK = 8
_NCHUNKS = _ROWS_PER_W // _CHUNK
_RSTRIDE = 56
_BV = 32768


def _tc_pad_transpose(table_t):
    def body(t_ref, out_ref):
        x = t_ref[...]
        eye = (lax.broadcasted_iota(jnp.int32, (_EMBED, _EMBED), 0) ==
               lax.broadcasted_iota(jnp.int32, (_EMBED, _EMBED), 1)
               ).astype(jnp.float32)
        xt = lax.dot_general(x, eye, (((0,), (0,)), ((), ())),
                             preferred_element_type=jnp.float32)
        out_ref[...] = jnp.concatenate(
            [xt, jnp.zeros((_BV, _PAD - _EMBED), jnp.float32)], axis=1)

    grid = (_VOCAB + _BV - 1) // _BV
    return pl.pallas_call(
        body,
        grid=(grid,),
        in_specs=[pl.BlockSpec((_EMBED, _BV), lambda i: (0, i))],
        out_specs=pl.BlockSpec((_BV, _PAD), lambda i: (i, 0)),
        out_shape=jax.ShapeDtypeStruct((_VOCAB, _PAD), jnp.float32),
    )(table_t)


def _sc_body(idx_hbm, table_hbm, out_hbm,
             idx_a, idx_b, rows_a, rows_b, out_v, sem_i, sem_g):
    wid = lax.axis_index("s") * _NC + lax.axis_index("c")
    row0 = wid * _ROWS_PER_W
    idx_bufs = (idx_a, idx_b)
    rows_bufs = (rows_a, rows_b)

    lanes = lax.iota(jnp.int32, 16)
    c_row = lanes & 7
    c_col0 = (lanes >> 3) * (_SUBLEN // 2)

    def chunk_base(chunk):
        return pl.multiple_of(row0 + chunk * _CHUNK, _CHUNK)

    def idx_src(chunk):
        return idx_hbm.at[pl.ds(chunk_base(chunk), _CHUNK)]

    def issue_idx(chunk, p):
        pltpu.async_copy(idx_src(chunk), idx_bufs[p], sem_i)

    def wait_idx(chunk, p):
        pltpu.make_async_copy(idx_src(chunk), idx_bufs[p], sem_i).wait()

    def gather_copies(p):
        return [
            pltpu.make_async_copy(
                table_hbm.at[idx_bufs[p].at[b, pl.ds(0, _SUBLEN)]],
                rows_bufs[p].at[pl.ds(b * _RSTRIDE, _SUBLEN)],
                sem_g)
            for b in range(_CHUNK)
        ]

    def issue_gathers(p):
        for cp in gather_copies(p):
            cp.start()

    def wait_gathers(p):
        for cp in gather_copies(p):
            cp.wait()

    def count_chunk(p):
        cnt = jnp.zeros((16,), jnp.float32)
        for s in range(_SUBLEN // 2):
            v = plsc.load_gather(idx_bufs[p], [c_row, c_col0 + s])
            cnt = cnt + jnp.where(v != 0, 1.0, 0.0)
        cnt_sw = cnt[(lanes + 8) & 15]
        return 1.0 / (cnt + cnt_sw)

    def compute(chunk, p, rcp):
        rows_v = rows_bufs[p]
        for b in range(_CHUNK):
            r = rcp[b]

            def sbody(s, accs, b=b):
                row = b * _RSTRIDE + s
                return tuple(
                    accs[d] + rows_v[row, pl.ds(d * 16, 16)] for d in range(4)
                )

            z = jnp.zeros((16,), jnp.float32)
            accs = lax.fori_loop(0, _SUBLEN, sbody, (z, z, z, z), unroll=10)
            for d in range(4):
                out_v[b, pl.ds(d * 16, 16)] = accs[d] * r
        pltpu.sync_copy(out_v, out_hbm.at[pl.ds(chunk_base(chunk), _CHUNK)])

    issue_idx(0, 0)
    wait_idx(0, 0)
    cnt0 = count_chunk(0)
    issue_gathers(0)
    issue_idx(1, 1)

    def body2(t, cnt_cur):
        for q in range(2):
            i = 2 * t + q
            wait_gathers(q)
            wait_idx(i + 1, 1 - q)
            cnt_next = count_chunk(1 - q)
            issue_gathers(1 - q)
            issue_idx(i + 2, q)
            compute(i, q, cnt_cur)
            cnt_cur = cnt_next
        return cnt_cur

    cnt_cur = lax.fori_loop(0, (_NCHUNKS - 2) // 2, body2, cnt0)

    wait_gathers(0)
    wait_idx(_NCHUNKS - 1, 1)
    cnt_last = count_chunk(1)
    issue_gathers(1)
    compute(_NCHUNKS - 2, 0, cnt_cur)
    wait_gathers(1)
    compute(_NCHUNKS - 1, 1, cnt_last)


def _sc_gather(idx, table_pad):
    mesh = plsc.VectorSubcoreMesh(core_axis_name="c", subcore_axis_name="s")
    return pl.kernel(
        _sc_body,
        out_type=jax.ShapeDtypeStruct((_BATCH, _EMBED), jnp.float32),
        mesh=mesh,
        scratch_types=[
            pltpu.VMEM((_CHUNK, _PAD), jnp.int32),
            pltpu.VMEM((_CHUNK, _PAD), jnp.int32),
            pltpu.VMEM((_CHUNK * _RSTRIDE, _PAD), jnp.float32),
            pltpu.VMEM((_CHUNK * _RSTRIDE, _PAD), jnp.float32),
            pltpu.VMEM((_CHUNK, _EMBED), jnp.float32),
            pltpu.SemaphoreType.DMA,
            pltpu.SemaphoreType.DMA,
        ],
        compiler_params=pltpu.CompilerParams(
            use_tc_tiling_on_sc=True, needs_layout_passes=False),
    )(idx, table_pad)


@jax.jit
def _sub_token_embed(idx, table_t):
    return _sc_gather(idx, _tc_pad_transpose(table_t))


def kernel(sub_tokens_indices, embeddings_weight):
    idx_p = jnp.pad(sub_tokens_indices.astype(jnp.int32),
                    ((0, 0), (0, _PAD - _SUBLEN)))
    return _sub_token_embed(idx_p, embeddings_weight.T)


# speedup vs baseline: 1.0119x; 1.0119x over previous
import functools

import jax
import jax.numpy as jnp
from jax import lax
from jax.experimental import pallas as pl
from jax.experimental.pallas import tpu as pltpu
from jax.experimental.pallas import tpu_sc as plsc

_BATCH = 16384
_SUBLEN = 50
_EMBED = 64
_VOCAB = 1000000
_PAD = 128
_NC = 2
_NS = 16
_NW = _NC * _NS
_ROWS_PER_W = _BATCH // _NW
_CHUNK = 8
_NCHUNKS = _ROWS_PER_W // _CHUNK
_RSTRIDE = 56
_BV = 32768


def _tc_pad_transpose(table_t):
    def body(t_ref, out_ref):
        x = t_ref[...]
        eye = (lax.broadcasted_iota(jnp.int32, (_EMBED, _EMBED), 0) ==
               lax.broadcasted_iota(jnp.int32, (_EMBED, _EMBED), 1)
               ).astype(jnp.float32)
        xt = lax.dot_general(x, eye, (((0,), (0,)), ((), ())),
                             preferred_element_type=jnp.float32)
        out_ref[...] = jnp.concatenate(
            [xt, jnp.zeros((_BV, _PAD - _EMBED), jnp.float32)], axis=1)

    grid = (_VOCAB + _BV - 1) // _BV
    return pl.pallas_call(
        body,
        grid=(grid,),
        in_specs=[pl.BlockSpec((_EMBED, _BV), lambda i: (0, i))],
        out_specs=pl.BlockSpec((_BV, _PAD), lambda i: (i, 0)),
        out_shape=jax.ShapeDtypeStruct((_VOCAB, _PAD), jnp.float32),
    )(table_t)


def _sc_body(idx_hbm, table_hbm, out_hbm,
             idx_a, idx_b, rows_a, rows_b, out_v, sem_i, sem_g):
    wid = lax.axis_index("s") * _NC + lax.axis_index("c")
    row0 = wid * _ROWS_PER_W
    idx_bufs = (idx_a, idx_b)
    rows_bufs = (rows_a, rows_b)

    lanes = lax.iota(jnp.int32, 16)
    c_row = lanes & 7
    c_col0 = (lanes >> 3) * (_SUBLEN // 2)

    def chunk_base(chunk):
        return pl.multiple_of(row0 + chunk * _CHUNK, _CHUNK)

    def idx_src(chunk):
        return idx_hbm.at[pl.ds(chunk_base(chunk), _CHUNK)]

    def issue_idx(chunk, p):
        pltpu.async_copy(idx_src(chunk), idx_bufs[p], sem_i)

    def wait_idx(chunk, p):
        pltpu.make_async_copy(idx_src(chunk), idx_bufs[p], sem_i).wait()

    def gather_copies(p):
        return [
            pltpu.make_async_copy(
                table_hbm.at[idx_bufs[p].at[b, pl.ds(0, _SUBLEN)]],
                rows_bufs[p].at[pl.ds(b * _RSTRIDE, _SUBLEN)],
                sem_g)
            for b in range(_CHUNK)
        ]

    def issue_gathers(p):
        for cp in gather_copies(p):
            cp.start()

    def wait_gathers(p):
        for cp in gather_copies(p):
            cp.wait()

    def count_chunk(p):
        cnt = jnp.zeros((16,), jnp.float32)
        for s in range(_SUBLEN // 2):
            v = plsc.load_gather(idx_bufs[p], [c_row, c_col0 + s])
            cnt = cnt + jnp.where(v != 0, 1.0, 0.0)
        cnt_sw = cnt[(lanes + 8) & 15]
        return 1.0 / (cnt + cnt_sw)

    def compute(chunk, p, rcp):
        rows_v = rows_bufs[p]
        for b in range(_CHUNK):
            r = rcp[b]

            def sbody(s, accs, b=b):
                row = b * _RSTRIDE + s
                return tuple(
                    accs[d] + rows_v[row, pl.ds(d * 16, 16)] for d in range(4)
                )

            z = jnp.zeros((16,), jnp.float32)
            accs = lax.fori_loop(0, _SUBLEN, sbody, (z, z, z, z), unroll=5)
            for d in range(4):
                out_v[b, pl.ds(d * 16, 16)] = accs[d] * r
        pltpu.sync_copy(out_v, out_hbm.at[pl.ds(chunk_base(chunk), _CHUNK)])

    issue_idx(0, 0)
    wait_idx(0, 0)
    cnt0 = count_chunk(0)
    issue_gathers(0)
    issue_idx(1, 1)

    def body2(t, cnt_cur):
        for q in range(2):
            i = 2 * t + q
            wait_gathers(q)
            wait_idx(i + 1, 1 - q)
            cnt_next = count_chunk(1 - q)
            issue_gathers(1 - q)
            issue_idx(i + 2, q)
            compute(i, q, cnt_cur)
            cnt_cur = cnt_next
        return cnt_cur

    cnt_cur = lax.fori_loop(0, (_NCHUNKS - 2) // 2, body2, cnt0)

    wait_gathers(0)
    wait_idx(_NCHUNKS - 1, 1)
    cnt_last = count_chunk(1)
    issue_gathers(1)
    compute(_NCHUNKS - 2, 0, cnt_cur)
    wait_gathers(1)
    compute(_NCHUNKS - 1, 1, cnt_last)


def _sc_gather(idx, table_pad):
    mesh = plsc.VectorSubcoreMesh(core_axis_name="c", subcore_axis_name="s")
    return pl.kernel(
        _sc_body,
        out_type=jax.ShapeDtypeStruct((_BATCH, _EMBED), jnp.float32),
        mesh=mesh,
        scratch_types=[
            pltpu.VMEM((_CHUNK, _PAD), jnp.int32),
            pltpu.VMEM((_CHUNK, _PAD), jnp.int32),
            pltpu.VMEM((_CHUNK * _RSTRIDE, _PAD), jnp.float32),
            pltpu.VMEM((_CHUNK * _RSTRIDE, _PAD), jnp.float32),
            pltpu.VMEM((_CHUNK, _EMBED), jnp.float32),
            pltpu.SemaphoreType.DMA,
            pltpu.SemaphoreType.DMA,
        ],
        compiler_params=pltpu.CompilerParams(
            use_tc_tiling_on_sc=True, needs_layout_passes=False),
    )(idx, table_pad)


@jax.jit
def _sub_token_embed(idx, table_t):
    return _sc_gather(idx, _tc_pad_transpose(table_t))


def kernel(sub_tokens_indices, embeddings_weight):
    idx_p = jnp.pad(sub_tokens_indices.astype(jnp.int32),
                    ((0, 0), (0, _PAD - _SUBLEN)))
    return _sub_token_embed(idx_p, embeddings_weight.T)
